# Initial kernel scaffold; baseline (speedup 1.0000x reference)
import jax
import jax.numpy as jnp
from jax.experimental import pallas as pl


def kernel(x, edge_index, batch, W1, b1, W2, b2, fc1_W, fc1_b, fc2_W, fc2_b):
    raise NotImplementedError("write your pallas kernel here")


# speedup vs baseline: 56.1176x; 56.1176x over previous
import functools

import jax
import jax.numpy as jnp
from jax import lax
from jax.experimental import pallas as pl
from jax.experimental.pallas import tpu as pltpu
from jax.experimental.pallas import tpu_sc as plsc

_CH = 128
_BLK = 8
_NW = 32
_G = 64
_PAD = 32


def _sc_degree(col2, np_):
    rows, ch = col2.shape
    rows_w = rows // _NW
    nblk = rows_w // _BLK
    mesh = plsc.VectorSubcoreMesh(core_axis_name="c", subcore_axis_name="s")

    @functools.partial(
        pl.kernel,
        out_type=jax.ShapeDtypeStruct((2, 1, np_), jnp.float32),
        mesh=mesh,
        scratch_types=[
            pltpu.VMEM((_BLK, ch), jnp.int32),
            pltpu.VMEM((ch,), jnp.float32),
            pltpu.VMEM_SHARED((np_,), jnp.float32),
            pltpu.SemaphoreType.DMA,
        ],
        compiler_params=pltpu.CompilerParams(use_tc_tiling_on_sc=False),
    )
    def deg_kernel(col_hbm, ones_hbm, z_hbm, out_hbm, colv, onesv, acc, sem):
        cid = lax.axis_index("c")
        sid = lax.axis_index("s")
        wid = cid * 16 + sid

        @pl.when(sid == 0)
        def _zero():
            pltpu.sync_copy(z_hbm, acc)

        pltpu.sync_copy(ones_hbm, onesv)
        plsc.subcore_barrier()

        base = wid * rows_w

        def body(b, carry):
            pltpu.sync_copy(col_hbm.at[pl.ds(base + b * _BLK, _BLK)], colv)
            ds = [
                pltpu.async_copy(onesv, acc.at[colv.at[j]], sem, add=True)
                for j in range(_BLK)
            ]
            for d in ds:
                d.wait()
            return carry

        lax.fori_loop(0, nblk, body, 0)
        plsc.subcore_barrier()

        @pl.when(sid == 0)
        def _out():
            pltpu.sync_copy(acc, out_hbm.at[cid].at[0])

    return deg_kernel(col2, jnp.ones((ch,), jnp.float32),
                      jnp.zeros((np_,), jnp.float32))


def _sc_scatter(row2, col2, u, np_):
    rows, ch = row2.shape
    n, f = u.shape
    rows_w = rows // _NW
    nblk = rows_w // _BLK
    mesh = plsc.VectorSubcoreMesh(core_axis_name="c", subcore_axis_name="s")

    @functools.partial(
        pl.kernel,
        out_type=jax.ShapeDtypeStruct((2, np_, f), jnp.float32),
        mesh=mesh,
        scratch_types=[
            pltpu.VMEM((_BLK, ch), jnp.int32),
            pltpu.VMEM((_BLK, ch), jnp.int32),
            pltpu.VMEM((_BLK * ch, f), jnp.float32),
            pltpu.VMEM_SHARED((np_, f), jnp.float32),
            pltpu.SemaphoreType.DMA,
            pltpu.SemaphoreType.DMA,
        ],
        compiler_params=pltpu.CompilerParams(use_tc_tiling_on_sc=False),
    )
    def scat_kernel(row_hbm, col_hbm, u_hbm, z_hbm, out_hbm,
                    rowv, colv, datav, acc, sem_g, sem_s):
        cid = lax.axis_index("c")
        sid = lax.axis_index("s")
        wid = cid * 16 + sid

        @pl.when(sid == 0)
        def _zero():
            pltpu.sync_copy(z_hbm, acc)

        plsc.subcore_barrier()

        base = wid * rows_w

        def body(b, carry):
            pltpu.sync_copy(row_hbm.at[pl.ds(base + b * _BLK, _BLK)], rowv)
            pltpu.sync_copy(col_hbm.at[pl.ds(base + b * _BLK, _BLK)], colv)
            gd = [
                pltpu.async_copy(u_hbm.at[rowv.at[j]],
                                 datav.at[pl.ds(j * ch, ch)], sem_g)
                for j in range(_BLK)
            ]
            for d in gd:
                d.wait()
            sd = [
                pltpu.async_copy(datav.at[pl.ds(j * ch, ch)],
                                 acc.at[colv.at[j]], sem_s, add=True)
                for j in range(_BLK)
            ]
            for d in sd:
                d.wait()
            return carry

        lax.fori_loop(0, nblk, body, 0)
        plsc.subcore_barrier()

        @pl.when(sid == 0)
        def _out():
            pltpu.sync_copy(acc, out_hbm.at[cid])

    return scat_kernel(row2, col2, u, jnp.zeros((np_, f), jnp.float32))


def _tc_prep(deg0, deg1, x, bn):
    n = x.shape[0]

    def body(d0_ref, d1_ref, x_ref, dis_ref, u1_ref):
        dis = lax.rsqrt(d0_ref[...] + d1_ref[...] + 1.0)
        dis_ref[...] = dis
        u1_ref[...] = jnp.pad(x_ref[...] * dis, ((0, 0), (0, 6)))

    return pl.pallas_call(
        body,
        grid=(n // bn,),
        in_specs=[
            pl.BlockSpec((bn, 1), lambda i: (i, 0)),
            pl.BlockSpec((bn, 1), lambda i: (i, 0)),
            pl.BlockSpec((bn, 2), lambda i: (i, 0)),
        ],
        out_specs=[
            pl.BlockSpec((bn, 1), lambda i: (i, 0)),
            pl.BlockSpec((bn, 8), lambda i: (i, 0)),
        ],
        out_shape=[
            jax.ShapeDtypeStruct((n, 1), jnp.float32),
            jax.ShapeDtypeStruct((n, 8), jnp.float32),
        ],
    )(deg0, deg1, x)


def _tc_layer1(a0, a1, dis, x, W1, b1, bn):
    n, f = a0.shape
    h = W1.shape[1]

    def body(a0_ref, a1_ref, dis_ref, x_ref, w_ref, b_ref, u2_ref):
        dis = dis_ref[...]
        u1 = x_ref[...] * dis
        t = dis * (a0_ref[...][:, :2] + a1_ref[...][:, :2] + u1)
        h1 = jnp.maximum(
            lax.dot_general(t, w_ref[...], (((1,), (0,)), ((), ())),
                            preferred_element_type=jnp.float32) + b_ref[...],
            0.0)
        u2_ref[...] = h1 * dis

    return pl.pallas_call(
        body,
        grid=(n // bn,),
        in_specs=[
            pl.BlockSpec((bn, 8), lambda i: (i, 0)),
            pl.BlockSpec((bn, 8), lambda i: (i, 0)),
            pl.BlockSpec((bn, 1), lambda i: (i, 0)),
            pl.BlockSpec((bn, 2), lambda i: (i, 0)),
            pl.BlockSpec((2, h), lambda i: (0, 0)),
            pl.BlockSpec((1, h), lambda i: (0, 0)),
        ],
        out_specs=pl.BlockSpec((bn, h), lambda i: (i, 0)),
        out_shape=jax.ShapeDtypeStruct((n, h), jnp.float32),
    )(a0, a1, dis, x, W1, b1)


def _tc_final(a0, a1, dis, u2, W2, b2, batch2, f1W, f1b, f2W, f2b, bn):
    n, h = a0.shape

    def body(a0_ref, a1_ref, dis_ref, u2_ref, w_ref, b_ref, bat_ref,
             f1w_ref, f1b_ref, f2w_ref, f2b_ref, out_ref, pooled, cnts):
        i = pl.program_id(0)

        @pl.when(i == 0)
        def _init():
            pooled[...] = jnp.zeros_like(pooled)
            cnts[...] = jnp.zeros_like(cnts)

        dis = dis_ref[...]
        t = dis * (a0_ref[...] + a1_ref[...] + u2_ref[...])
        h2 = jnp.maximum(
            lax.dot_general(t, w_ref[...], (((1,), (0,)), ((), ())),
                            preferred_element_type=jnp.float32) + b_ref[...],
            0.0)
        gids = lax.broadcasted_iota(jnp.int32, (bn, _G), 1)
        onehot = (bat_ref[...] == gids).astype(jnp.float32)
        pooled[...] += lax.dot_general(
            onehot, h2, (((0,), (0,)), ((), ())),
            preferred_element_type=jnp.float32)
        cnts[...] += lax.dot_general(
            onehot, jnp.ones((bn, h), jnp.float32), (((0,), (0,)), ((), ())),
            preferred_element_type=jnp.float32)

        @pl.when(i == pl.num_programs(0) - 1)
        def _fin():
            mean = pooled[...] / jnp.maximum(cnts[...], 1.0)
            hm = jnp.maximum(
                lax.dot_general(mean, f1w_ref[...], (((1,), (0,)), ((), ())),
                                preferred_element_type=jnp.float32)
                + f1b_ref[...], 0.0)
            out_ref[...] = lax.dot_general(
                hm, f2w_ref[...], (((1,), (0,)), ((), ())),
                preferred_element_type=jnp.float32) + f2b_ref[...]

    return pl.pallas_call(
        body,
        grid=(n // bn,),
        in_specs=[
            pl.BlockSpec((bn, h), lambda i: (i, 0)),
            pl.BlockSpec((bn, h), lambda i: (i, 0)),
            pl.BlockSpec((bn, 1), lambda i: (i, 0)),
            pl.BlockSpec((bn, h), lambda i: (i, 0)),
            pl.BlockSpec((h, h), lambda i: (0, 0)),
            pl.BlockSpec((1, h), lambda i: (0, 0)),
            pl.BlockSpec((bn, 1), lambda i: (i, 0)),
            pl.BlockSpec((h, h), lambda i: (0, 0)),
            pl.BlockSpec((1, h), lambda i: (0, 0)),
            pl.BlockSpec((h, 1), lambda i: (0, 0)),
            pl.BlockSpec((1, 1), lambda i: (0, 0)),
        ],
        out_specs=pl.BlockSpec((_G, 1), lambda i: (0, 0)),
        out_shape=jax.ShapeDtypeStruct((_G, 1), jnp.float32),
        scratch_shapes=[
            pltpu.VMEM((_G, h), jnp.float32),
            pltpu.VMEM((_G, h), jnp.float32),
        ],
    )(a0, a1, dis, u2, W2, b2, batch2, f1W, f1b, f2W, f2b)


def kernel(x, edge_index, batch, W1, b1, W2, b2, fc1_W, fc1_b, fc2_W, fc2_b):
    n = x.shape[0]
    e = edge_index.shape[1]
    bn = 10000
    np_ = n + _PAD

    quant = _NW * _BLK * _CH
    ep = ((e + quant - 1) // quant) * quant
    pad = ep - e
    spread = (jnp.arange(pad, dtype=jnp.int32) % _PAD)
    rowp = jnp.concatenate([edge_index[0], spread])
    colp = jnp.concatenate([edge_index[1], n + spread])
    row2 = rowp.reshape(ep // _CH, _CH)
    col2 = colp.reshape(ep // _CH, _CH)

    degp = _sc_degree(col2, np_)
    deg0 = degp[0, 0, :n].reshape(n, 1)
    deg1 = degp[1, 0, :n].reshape(n, 1)
    dis, u1 = _tc_prep(deg0, deg1, x, bn)

    agg1 = _sc_scatter(row2, col2, u1, np_)
    u2 = _tc_layer1(agg1[0, :n], agg1[1, :n], dis, x, W1,
                    b1.reshape(1, -1), bn)

    agg2 = _sc_scatter(row2, col2, u2, np_)
    out = _tc_final(agg2[0, :n], agg2[1, :n], dis, u2, W2,
                    b2.reshape(1, -1), batch.reshape(n, 1),
                    fc1_W, fc1_b.reshape(1, -1), fc2_W,
                    fc2_b.reshape(1, 1), bn)
    return out


# speedup vs baseline: 60.6590x; 1.0809x over previous
import functools

import jax
import jax.numpy as jnp
from jax import lax
from jax.experimental import pallas as pl
from jax.experimental.pallas import tpu as pltpu
from jax.experimental.pallas import tpu_sc as plsc

_CH = 128
_BLK = 8
_NW = 32
_G = 64


def _sc_degree(ei3, n):
    _, rows, ch = ei3.shape
    nbt = rows // _BLK
    nbw = (nbt + _NW - 1) // _NW
    nbw = nbw + (nbw % 2)
    mesh = plsc.VectorSubcoreMesh(core_axis_name="c", subcore_axis_name="s")

    @functools.partial(
        pl.kernel,
        out_type=jax.ShapeDtypeStruct((2, 1, n), jnp.float32),
        mesh=mesh,
        scratch_types=[
            pltpu.VMEM((_BLK, ch), jnp.int32),
            pltpu.VMEM((_BLK, ch), jnp.int32),
            pltpu.VMEM((ch,), jnp.float32),
            pltpu.VMEM_SHARED((n,), jnp.float32),
            pltpu.SemaphoreType.DMA,
            pltpu.SemaphoreType.DMA,
        ],
        compiler_params=pltpu.CompilerParams(use_tc_tiling_on_sc=False),
    )
    def deg_kernel(ei_hbm, ones_hbm, z_hbm, out_hbm,
                   colv0, colv1, onesv, acc, sem0, sem1):
        cid = lax.axis_index("c")
        sid = lax.axis_index("s")
        wid = cid * 16 + sid
        col_hbm = ei_hbm.at[1]

        @pl.when(sid == 0)
        def _zero():
            pltpu.sync_copy(z_hbm, acc)

        pltpu.sync_copy(ones_hbm, onesv)
        plsc.subcore_barrier()

        def fire(colv, sem):
            for j in range(_BLK):
                pltpu.async_copy(onesv, acc.at[colv.at[j]], sem, add=True)

        def wait(colv, sem):
            for j in range(_BLK):
                pltpu.make_async_copy(onesv, acc.at[colv.at[j]], sem).wait()

        def body(g, carry):
            be = (2 * g) * _NW + wid
            bo = be + _NW

            @pl.when((bo - 2 * _NW >= 0) & (bo - 2 * _NW < nbt))
            def _():
                wait(colv1, sem1)

            @pl.when(be < nbt)
            def _():
                pltpu.sync_copy(col_hbm.at[pl.ds(be * _BLK, _BLK)], colv0)
                fire(colv0, sem0)

            @pl.when(bo < nbt)
            def _():
                pltpu.sync_copy(col_hbm.at[pl.ds(bo * _BLK, _BLK)], colv1)
                fire(colv1, sem1)

            @pl.when(be < nbt)
            def _():
                wait(colv0, sem0)

            return carry

        lax.fori_loop(0, nbw // 2, body, 0)
        blast = (nbw - 1) * _NW + wid

        @pl.when(blast < nbt)
        def _():
            wait(colv1, sem1)

        plsc.subcore_barrier()

        @pl.when(sid == 0)
        def _out():
            pltpu.sync_copy(acc, out_hbm.at[cid].at[0])

    return deg_kernel(ei3, jnp.ones((ch,), jnp.float32),
                      jnp.zeros((n,), jnp.float32))


def _sc_scatter(ei3, u):
    _, rows, ch = ei3.shape
    n, f = u.shape
    nbt = rows // _BLK
    nbw = (nbt + _NW - 1) // _NW
    nbw = nbw + (nbw % 2)
    hb = _BLK // 2
    mesh = plsc.VectorSubcoreMesh(core_axis_name="c", subcore_axis_name="s")

    @functools.partial(
        pl.kernel,
        out_type=jax.ShapeDtypeStruct((2, n, f), jnp.float32),
        mesh=mesh,
        scratch_types=[
            pltpu.VMEM((_BLK, ch), jnp.int32),
            pltpu.VMEM((_BLK, ch), jnp.int32),
            pltpu.VMEM((_BLK, ch), jnp.int32),
            pltpu.VMEM((_BLK, ch), jnp.int32),
            pltpu.VMEM((hb * ch, f), jnp.float32),
            pltpu.VMEM((hb * ch, f), jnp.float32),
            pltpu.VMEM_SHARED((n, f), jnp.float32),
            pltpu.SemaphoreType.DMA,
            pltpu.SemaphoreType.DMA,
            pltpu.SemaphoreType.DMA,
            pltpu.SemaphoreType.DMA,
        ],
        compiler_params=pltpu.CompilerParams(use_tc_tiling_on_sc=False),
    )
    def scat_kernel(ei_hbm, u_hbm, z_hbm, out_hbm,
                    rowv0, colv0, rowv1, colv1, data0, data1, acc,
                    semg0, semg1, sems0, sems1):
        cid = lax.axis_index("c")
        sid = lax.axis_index("s")
        wid = cid * 16 + sid
        row_hbm = ei_hbm.at[0]
        col_hbm = ei_hbm.at[1]

        @pl.when(sid == 0)
        def _zero():
            pltpu.sync_copy(z_hbm, acc)

        plsc.subcore_barrier()

        def load(b, rowv, colv):
            pltpu.sync_copy(row_hbm.at[pl.ds(b * _BLK, _BLK)], rowv)
            pltpu.sync_copy(col_hbm.at[pl.ds(b * _BLK, _BLK)], colv)

        def fire_g(rowv, j0, datav, sem):
            for j in range(hb):
                pltpu.async_copy(u_hbm.at[rowv.at[j0 + j]],
                                 datav.at[pl.ds(j * ch, ch)], sem)

        def wait_g(rowv, j0, datav, sem):
            for j in range(hb):
                pltpu.make_async_copy(
                    u_hbm.at[rowv.at[j0 + j]],
                    datav.at[pl.ds(j * ch, ch)], sem).wait()

        def fire_s(colv, j0, datav, sem):
            for j in range(hb):
                pltpu.async_copy(datav.at[pl.ds(j * ch, ch)],
                                 acc.at[colv.at[j0 + j]], sem, add=True)

        def wait_s(colv, j0, datav, sem):
            for j in range(hb):
                pltpu.make_async_copy(
                    datav.at[pl.ds(j * ch, ch)],
                    acc.at[colv.at[j0 + j]], sem).wait()

        def body(g, carry):
            be = (2 * g) * _NW + wid
            bo = be + _NW
            bp = bo - 2 * _NW
            cp = (bp >= 0) & (bp < nbt)
            ce = be < nbt
            co = bo < nbt

            @pl.when(cp)
            def _():
                wait_s(colv1, 0, data0, sems0)

            @pl.when(ce)
            def _():
                load(be, rowv0, colv0)
                fire_g(rowv0, 0, data0, semg0)

            @pl.when(cp)
            def _():
                wait_s(colv1, hb, data1, sems1)

            @pl.when(ce)
            def _():
                fire_g(rowv0, hb, data1, semg1)
                wait_g(rowv0, 0, data0, semg0)
                fire_s(colv0, 0, data0, sems0)

            @pl.when(co)
            def _():
                load(bo, rowv1, colv1)

            @pl.when(ce)
            def _():
                wait_g(rowv0, hb, data1, semg1)
                fire_s(colv0, hb, data1, sems1)

            @pl.when(co)
            def _():
                wait_s(colv0, 0, data0, sems0)
                fire_g(rowv1, 0, data0, semg0)
                wait_g(rowv1, 0, data0, semg0)
                fire_s(colv1, 0, data0, sems0)
                wait_s(colv0, hb, data1, sems1)
                fire_g(rowv1, hb, data1, semg1)
                wait_g(rowv1, hb, data1, semg1)
                fire_s(colv1, hb, data1, sems1)

            return carry

        lax.fori_loop(0, nbw // 2, body, 0)
        bo_last = (nbw - 1) * _NW + wid
        ce_last = (bo_last - _NW) < nbt
        co_last = bo_last < nbt

        @pl.when(co_last)
        def _():
            wait_s(colv1, 0, data0, sems0)
            wait_s(colv1, hb, data1, sems1)

        @pl.when(ce_last & jnp.logical_not(co_last))
        def _():
            wait_s(colv0, 0, data0, sems0)
            wait_s(colv0, hb, data1, sems1)

        plsc.subcore_barrier()

        @pl.when(sid == 0)
        def _out():
            pltpu.sync_copy(acc, out_hbm.at[cid])

    return scat_kernel(ei3, u, jnp.zeros((n, f), jnp.float32))


def _tc_prep(deg0, deg1, x, bn):
    n = x.shape[0]

    def body(d0_ref, d1_ref, x_ref, dis_ref, u1_ref):
        dis = lax.rsqrt(d0_ref[...] + d1_ref[...] + 1.0)
        dis_ref[...] = dis
        u1_ref[...] = jnp.pad(x_ref[...] * dis, ((0, 0), (0, 6)))

    return pl.pallas_call(
        body,
        grid=(n // bn,),
        in_specs=[
            pl.BlockSpec((bn, 1), lambda i: (i, 0)),
            pl.BlockSpec((bn, 1), lambda i: (i, 0)),
            pl.BlockSpec((bn, 2), lambda i: (i, 0)),
        ],
        out_specs=[
            pl.BlockSpec((bn, 1), lambda i: (i, 0)),
            pl.BlockSpec((bn, 8), lambda i: (i, 0)),
        ],
        out_shape=[
            jax.ShapeDtypeStruct((n, 1), jnp.float32),
            jax.ShapeDtypeStruct((n, 8), jnp.float32),
        ],
    )(deg0, deg1, x)


def _tc_layer1(a0, a1, dis, x, W1, b1, bn):
    n = a0.shape[0]
    h = W1.shape[1]

    def body(a0_ref, a1_ref, dis_ref, x_ref, w_ref, b_ref, u2_ref):
        dis = dis_ref[...]
        u1 = x_ref[...] * dis
        t = dis * (a0_ref[...][:, :2] + a1_ref[...][:, :2] + u1)
        h1 = jnp.maximum(
            lax.dot_general(t, w_ref[...], (((1,), (0,)), ((), ())),
                            preferred_element_type=jnp.float32) + b_ref[...],
            0.0)
        u2_ref[...] = h1 * dis

    return pl.pallas_call(
        body,
        grid=(n // bn,),
        in_specs=[
            pl.BlockSpec((bn, 8), lambda i: (i, 0)),
            pl.BlockSpec((bn, 8), lambda i: (i, 0)),
            pl.BlockSpec((bn, 1), lambda i: (i, 0)),
            pl.BlockSpec((bn, 2), lambda i: (i, 0)),
            pl.BlockSpec((2, h), lambda i: (0, 0)),
            pl.BlockSpec((1, h), lambda i: (0, 0)),
        ],
        out_specs=pl.BlockSpec((bn, h), lambda i: (i, 0)),
        out_shape=jax.ShapeDtypeStruct((n, h), jnp.float32),
    )(a0, a1, dis, x, W1, b1)


def _tc_final(a0, a1, dis, u2, W2, b2, batch2, f1W, f1b, f2W, f2b, bn):
    n, h = a0.shape

    def body(a0_ref, a1_ref, dis_ref, u2_ref, w_ref, b_ref, bat_ref,
             f1w_ref, f1b_ref, f2w_ref, f2b_ref, out_ref, pooled, cnts):
        i = pl.program_id(0)

        @pl.when(i == 0)
        def _init():
            pooled[...] = jnp.zeros_like(pooled)
            cnts[...] = jnp.zeros_like(cnts)

        dis = dis_ref[...]
        t = dis * (a0_ref[...] + a1_ref[...] + u2_ref[...])
        h2 = jnp.maximum(
            lax.dot_general(t, w_ref[...], (((1,), (0,)), ((), ())),
                            preferred_element_type=jnp.float32) + b_ref[...],
            0.0)
        gids = lax.broadcasted_iota(jnp.int32, (bn, _G), 1)
        onehot = (bat_ref[...] == gids).astype(jnp.float32)
        pooled[...] += lax.dot_general(
            onehot, h2, (((0,), (0,)), ((), ())),
            preferred_element_type=jnp.float32)
        cnts[...] += lax.dot_general(
            onehot, jnp.ones((bn, h), jnp.float32), (((0,), (0,)), ((), ())),
            preferred_element_type=jnp.float32)

        @pl.when(i == pl.num_programs(0) - 1)
        def _fin():
            mean = pooled[...] / jnp.maximum(cnts[...], 1.0)
            hm = jnp.maximum(
                lax.dot_general(mean, f1w_ref[...], (((1,), (0,)), ((), ())),
                                preferred_element_type=jnp.float32)
                + f1b_ref[...], 0.0)
            out_ref[...] = lax.dot_general(
                hm, f2w_ref[...], (((1,), (0,)), ((), ())),
                preferred_element_type=jnp.float32) + f2b_ref[...]

    return pl.pallas_call(
        body,
        grid=(n // bn,),
        in_specs=[
            pl.BlockSpec((bn, h), lambda i: (i, 0)),
            pl.BlockSpec((bn, h), lambda i: (i, 0)),
            pl.BlockSpec((bn, 1), lambda i: (i, 0)),
            pl.BlockSpec((bn, h), lambda i: (i, 0)),
            pl.BlockSpec((h, h), lambda i: (0, 0)),
            pl.BlockSpec((1, h), lambda i: (0, 0)),
            pl.BlockSpec((bn, 1), lambda i: (i, 0)),
            pl.BlockSpec((h, h), lambda i: (0, 0)),
            pl.BlockSpec((1, h), lambda i: (0, 0)),
            pl.BlockSpec((h, 1), lambda i: (0, 0)),
            pl.BlockSpec((1, 1), lambda i: (0, 0)),
        ],
        out_specs=pl.BlockSpec((_G, 1), lambda i: (0, 0)),
        out_shape=jax.ShapeDtypeStruct((_G, 1), jnp.float32),
        scratch_shapes=[
            pltpu.VMEM((_G, h), jnp.float32),
            pltpu.VMEM((_G, h), jnp.float32),
        ],
    )(a0, a1, dis, u2, W2, b2, batch2, f1W, f1b, f2W, f2b)


def kernel(x, edge_index, batch, W1, b1, W2, b2, fc1_W, fc1_b, fc2_W, fc2_b):
    n = x.shape[0]
    e = edge_index.shape[1]
    bn = 10000

    ei3 = edge_index.reshape(2, e // _CH, _CH)

    degp = _sc_degree(ei3, n)
    deg0 = degp[0, 0].reshape(n, 1)
    deg1 = degp[1, 0].reshape(n, 1)
    dis, u1 = _tc_prep(deg0, deg1, x, bn)

    agg1 = _sc_scatter(ei3, u1)
    u2 = _tc_layer1(agg1[0], agg1[1], dis, x, W1, b1.reshape(1, -1), bn)

    agg2 = _sc_scatter(ei3, u2)
    out = _tc_final(agg2[0], agg2[1], dis, u2, W2, b2.reshape(1, -1),
                    batch.reshape(n, 1), fc1_W, fc1_b.reshape(1, -1),
                    fc2_W, fc2_b.reshape(1, 1), bn)
    return out


# speedup vs baseline: 63.0509x; 1.0394x over previous
import functools

import jax
import jax.numpy as jnp
from jax import lax
from jax.experimental import pallas as pl
from jax.experimental.pallas import tpu as pltpu
from jax.experimental.pallas import tpu_sc as plsc

_CH = 128
_BLK = 8
_NW = 32
_G = 64
_H = 16
_GRID = 49


def _sc_degree(ei3, n, npad):
    _, rows, ch = ei3.shape
    nbt = rows // _BLK
    nbw = (nbt + _NW - 1) // _NW
    nbw = nbw + (nbw % 2)
    mesh = plsc.VectorSubcoreMesh(core_axis_name="c", subcore_axis_name="s")

    @functools.partial(
        pl.kernel,
        out_type=jax.ShapeDtypeStruct((2, 1, npad), jnp.float32),
        mesh=mesh,
        scratch_types=[
            pltpu.VMEM((_BLK, ch), jnp.int32),
            pltpu.VMEM((_BLK, ch), jnp.int32),
            pltpu.VMEM((ch,), jnp.float32),
            pltpu.VMEM_SHARED((n,), jnp.float32),
            pltpu.SemaphoreType.DMA,
            pltpu.SemaphoreType.DMA,
        ],
        compiler_params=pltpu.CompilerParams(use_tc_tiling_on_sc=False),
    )
    def deg_kernel(ei_hbm, ones_hbm, z_hbm, out_hbm,
                   colv0, colv1, onesv, acc, sem0, sem1):
        cid = lax.axis_index("c")
        sid = lax.axis_index("s")
        wid = cid * 16 + sid
        col_hbm = ei_hbm.at[1]

        @pl.when(sid == 0)
        def _zero():
            pltpu.sync_copy(z_hbm, acc)

        pltpu.sync_copy(ones_hbm, onesv)
        plsc.subcore_barrier()

        def fire(colv, sem):
            for j in range(_BLK):
                pltpu.async_copy(onesv, acc.at[colv.at[j]], sem, add=True)

        def wait(colv, sem):
            for j in range(_BLK):
                pltpu.make_async_copy(onesv, acc.at[colv.at[j]], sem).wait()

        def body(g, carry):
            be = (2 * g) * _NW + wid
            bo = be + _NW

            @pl.when((bo - 2 * _NW >= 0) & (bo - 2 * _NW < nbt))
            def _():
                wait(colv1, sem1)

            @pl.when(be < nbt)
            def _():
                pltpu.sync_copy(col_hbm.at[pl.ds(be * _BLK, _BLK)], colv0)
                fire(colv0, sem0)

            @pl.when(bo < nbt)
            def _():
                pltpu.sync_copy(col_hbm.at[pl.ds(bo * _BLK, _BLK)], colv1)
                fire(colv1, sem1)

            @pl.when(be < nbt)
            def _():
                wait(colv0, sem0)

            return carry

        lax.fori_loop(0, nbw // 2, body, 0)
        blast = (nbw - 1) * _NW + wid

        @pl.when(blast < nbt)
        def _():
            wait(colv1, sem1)

        plsc.subcore_barrier()

        @pl.when(sid == 0)
        def _out():
            pltpu.sync_copy(acc, out_hbm.at[cid, 0, pl.ds(0, n)])

    return deg_kernel(ei3, jnp.ones((ch,), jnp.float32),
                      jnp.zeros((n,), jnp.float32))


def _sc_scatter(ei3, u, n):
    _, rows, ch = ei3.shape
    npad, f = u.shape
    nbt = rows // _BLK
    nbw = (nbt + _NW - 1) // _NW
    nbw = nbw + (nbw % 2)
    hb = _BLK // 2
    mesh = plsc.VectorSubcoreMesh(core_axis_name="c", subcore_axis_name="s")

    @functools.partial(
        pl.kernel,
        out_type=jax.ShapeDtypeStruct((2, npad, f), jnp.float32),
        mesh=mesh,
        scratch_types=[
            pltpu.VMEM((_BLK, ch), jnp.int32),
            pltpu.VMEM((_BLK, ch), jnp.int32),
            pltpu.VMEM((_BLK, ch), jnp.int32),
            pltpu.VMEM((_BLK, ch), jnp.int32),
            pltpu.VMEM((_BLK // 2 * ch, f), jnp.float32),
            pltpu.VMEM((_BLK // 2 * ch, f), jnp.float32),
            pltpu.VMEM_SHARED((n, f), jnp.float32),
            pltpu.SemaphoreType.DMA,
            pltpu.SemaphoreType.DMA,
            pltpu.SemaphoreType.DMA,
            pltpu.SemaphoreType.DMA,
        ],
        compiler_params=pltpu.CompilerParams(use_tc_tiling_on_sc=False),
    )
    def scat_kernel(ei_hbm, u_hbm, z_hbm, out_hbm,
                    rowv0, colv0, rowv1, colv1, data0, data1, acc,
                    semg0, semg1, sems0, sems1):
        hb_ = _BLK // 2
        cid = lax.axis_index("c")
        sid = lax.axis_index("s")
        wid = cid * 16 + sid
        row_hbm = ei_hbm.at[0]
        col_hbm = ei_hbm.at[1]

        @pl.when(sid == 0)
        def _zero():
            pltpu.sync_copy(z_hbm, acc)

        plsc.subcore_barrier()

        def load(b, rowv, colv):
            pltpu.sync_copy(row_hbm.at[pl.ds(b * _BLK, _BLK)], rowv)
            pltpu.sync_copy(col_hbm.at[pl.ds(b * _BLK, _BLK)], colv)

        def fire_g(rowv, j0, datav, sem):
            for j in range(hb_):
                pltpu.async_copy(u_hbm.at[rowv.at[j0 + j]],
                                 datav.at[pl.ds(j * ch, ch)], sem)

        def wait_g(rowv, j0, datav, sem):
            for j in range(hb_):
                pltpu.make_async_copy(
                    u_hbm.at[rowv.at[j0 + j]],
                    datav.at[pl.ds(j * ch, ch)], sem).wait()

        def fire_s(colv, j0, datav, sem):
            for j in range(hb_):
                pltpu.async_copy(datav.at[pl.ds(j * ch, ch)],
                                 acc.at[colv.at[j0 + j]], sem, add=True)

        def wait_s(colv, j0, datav, sem):
            for j in range(hb_):
                pltpu.make_async_copy(
                    datav.at[pl.ds(j * ch, ch)],
                    acc.at[colv.at[j0 + j]], sem).wait()

        def body(g, carry):
            be = (2 * g) * _NW + wid
            bo = be + _NW
            bp = bo - 2 * _NW
            cp = (bp >= 0) & (bp < nbt)
            ce = be < nbt
            co = bo < nbt

            @pl.when(cp)
            def _():
                wait_s(colv1, 0, data0, sems0)

            @pl.when(ce)
            def _():
                load(be, rowv0, colv0)
                fire_g(rowv0, 0, data0, semg0)

            @pl.when(cp)
            def _():
                wait_s(colv1, hb_, data1, sems1)

            @pl.when(ce)
            def _():
                fire_g(rowv0, hb_, data1, semg1)
                wait_g(rowv0, 0, data0, semg0)
                fire_s(colv0, 0, data0, sems0)

            @pl.when(co)
            def _():
                load(bo, rowv1, colv1)

            @pl.when(ce)
            def _():
                wait_g(rowv0, hb_, data1, semg1)
                fire_s(colv0, hb_, data1, sems1)

            @pl.when(co)
            def _():
                wait_s(colv0, 0, data0, sems0)
                fire_g(rowv1, 0, data0, semg0)
                wait_g(rowv1, 0, data0, semg0)
                fire_s(colv1, 0, data0, sems0)
                wait_s(colv0, hb_, data1, sems1)
                fire_g(rowv1, hb_, data1, semg1)
                wait_g(rowv1, hb_, data1, semg1)
                fire_s(colv1, hb_, data1, sems1)

            return carry

        lax.fori_loop(0, nbw // 2, body, 0)
        bo_last = (nbw - 1) * _NW + wid
        ce_last = (bo_last - _NW) < nbt
        co_last = bo_last < nbt

        @pl.when(co_last)
        def _():
            wait_s(colv1, 0, data0, sems0)
            wait_s(colv1, hb_, data1, sems1)

        @pl.when(ce_last & jnp.logical_not(co_last))
        def _():
            wait_s(colv0, 0, data0, sems0)
            wait_s(colv0, hb_, data1, sems1)

        plsc.subcore_barrier()

        @pl.when(sid == 0)
        def _out():
            pltpu.sync_copy(acc, out_hbm.at[cid, pl.ds(0, n)])

    return scat_kernel(ei3, u, jnp.zeros((n, f), jnp.float32))


def _tc_prep(d0w, d1w, xw):
    r = d0w.shape[0]
    rb = r // _GRID

    def body(d0_ref, d1_ref, x_ref, dis_ref, u1_ref):
        dis = lax.rsqrt(d0_ref[...] + d1_ref[...] + 1.0)
        dis_ref[...] = dis
        u1_ref[...] = x_ref[...] * dis

    return pl.pallas_call(
        body,
        grid=(_GRID,),
        in_specs=[pl.BlockSpec((rb, 128), lambda i: (i, 0))] * 3,
        out_specs=[pl.BlockSpec((rb, 128), lambda i: (i, 0))] * 2,
        out_shape=[jax.ShapeDtypeStruct((r, 128), jnp.float32)] * 2,
    )(d0w, d1w, xw)


def _tc_layer(a0w, a1w, uw, dis16w, bigW, biasw):
    r = a0w.shape[0]
    rb = r // _GRID

    def body(a0_ref, a1_ref, u_ref, dis_ref, w_ref, b_ref, o_ref):
        dis = dis_ref[...]
        t = dis * (a0_ref[...] + a1_ref[...] + u_ref[...])
        h = jnp.maximum(
            lax.dot_general(t, w_ref[...], (((1,), (0,)), ((), ())),
                            preferred_element_type=jnp.float32) + b_ref[...],
            0.0)
        o_ref[...] = h * dis

    return pl.pallas_call(
        body,
        grid=(_GRID,),
        in_specs=[
            pl.BlockSpec((rb, 128), lambda i: (i, 0)),
            pl.BlockSpec((rb, 128), lambda i: (i, 0)),
            pl.BlockSpec((rb, 128), lambda i: (i, 0)),
            pl.BlockSpec((rb, 128), lambda i: (i, 0)),
            pl.BlockSpec((128, 128), lambda i: (0, 0)),
            pl.BlockSpec((1, 128), lambda i: (0, 0)),
        ],
        out_specs=pl.BlockSpec((rb, 128), lambda i: (i, 0)),
        out_shape=jax.ShapeDtypeStruct((r, 128), jnp.float32),
    )(a0w, a1w, uw, dis16w, bigW, biasw)


def _tc_final(a0w, a1w, u2w, dis16w, bigW, biasw, bat8, efold,
              f1W, f1b, f2W, f2b):
    r = a0w.shape[0]
    rb = r // _GRID

    def body(a0_ref, a1_ref, u_ref, dis_ref, w_ref, b_ref, bat_ref,
             ef_ref, f1w_ref, f1b_ref, f2w_ref, f2b_ref, out_ref,
             pooled, cnts):
        i = pl.program_id(0)

        @pl.when(i == 0)
        def _init():
            pooled[...] = jnp.zeros_like(pooled)
            cnts[...] = jnp.zeros_like(cnts)

        dis = dis_ref[...]
        t = dis * (a0_ref[...] + a1_ref[...] + u_ref[...])
        h2 = jnp.maximum(
            lax.dot_general(t, w_ref[...], (((1,), (0,)), ((), ())),
                            preferred_element_type=jnp.float32) + b_ref[...],
            0.0)
        h2 = jnp.where(jnp.abs(h2) < 1e30, h2, 0.0)

        gcol = lax.broadcasted_iota(jnp.int32, (_G, rb), 0)
        slot = lax.broadcasted_iota(jnp.int32, (_G, 128), 1) // _H
        ones = jnp.ones((rb, 128), jnp.float32)
        psum = jnp.zeros((_G, 128), jnp.float32)
        csum = jnp.zeros((_G, 128), jnp.float32)
        for a in range(8):
            ba = bat_ref[a:a + 1, :]
            oh = (jnp.broadcast_to(ba, (_G, rb)) == gcol).astype(jnp.float32)
            m = (slot == a).astype(jnp.float32)
            psum += lax.dot_general(oh, h2, (((1,), (0,)), ((), ())),
                                    preferred_element_type=jnp.float32) * m
            csum += lax.dot_general(oh, ones, (((1,), (0,)), ((), ())),
                                    preferred_element_type=jnp.float32) * m

        ef = ef_ref[...]
        pooled[...] += lax.dot_general(psum, ef, (((1,), (0,)), ((), ())),
                                       preferred_element_type=jnp.float32)
        cnts[...] += lax.dot_general(csum, ef, (((1,), (0,)), ((), ())),
                                     preferred_element_type=jnp.float32)

        @pl.when(i == pl.num_programs(0) - 1)
        def _fin():
            mean = pooled[...] / jnp.maximum(cnts[...], 1.0)
            hm = jnp.maximum(
                lax.dot_general(mean, f1w_ref[...], (((1,), (0,)), ((), ())),
                                preferred_element_type=jnp.float32)
                + f1b_ref[...], 0.0)
            out_ref[...] = lax.dot_general(
                hm, f2w_ref[...], (((1,), (0,)), ((), ())),
                preferred_element_type=jnp.float32) + f2b_ref[...]

    return pl.pallas_call(
        body,
        grid=(_GRID,),
        in_specs=[
            pl.BlockSpec((rb, 128), lambda i: (i, 0)),
            pl.BlockSpec((rb, 128), lambda i: (i, 0)),
            pl.BlockSpec((rb, 128), lambda i: (i, 0)),
            pl.BlockSpec((rb, 128), lambda i: (i, 0)),
            pl.BlockSpec((128, 128), lambda i: (0, 0)),
            pl.BlockSpec((1, 128), lambda i: (0, 0)),
            pl.BlockSpec((8, rb), lambda i: (0, i)),
            pl.BlockSpec((128, _H), lambda i: (0, 0)),
            pl.BlockSpec((_H, _H), lambda i: (0, 0)),
            pl.BlockSpec((1, _H), lambda i: (0, 0)),
            pl.BlockSpec((_H, 1), lambda i: (0, 0)),
            pl.BlockSpec((1, 1), lambda i: (0, 0)),
        ],
        out_specs=pl.BlockSpec((_G, 1), lambda i: (0, 0)),
        out_shape=jax.ShapeDtypeStruct((_G, 1), jnp.float32),
        scratch_shapes=[
            pltpu.VMEM((_G, _H), jnp.float32),
            pltpu.VMEM((_G, _H), jnp.float32),
        ],
    )(a0w, a1w, u2w, dis16w, bigW, biasw, bat8, efold, f1W, f1b, f2W, f2b)


def _widen16(v, npad):
    return jnp.broadcast_to(v[:, None], (npad, _H)).reshape(npad // 8, 128)


def _blockdiag(W, b):
    Wp = jnp.zeros((_H, _H), jnp.float32).at[:W.shape[0], :].set(W)
    bigW = jnp.kron(jnp.eye(8, dtype=jnp.float32), Wp)
    biasw = jnp.tile(b, 8).reshape(1, 128)
    return bigW, biasw


def kernel(x, edge_index, batch, W1, b1, W2, b2, fc1_W, fc1_b, fc2_W, fc2_b):
    n = x.shape[0]
    e = edge_index.shape[1]
    npad = ((n + 1023) // 1024) * 1024
    r = npad // 8

    ei3 = edge_index.reshape(2, e // _CH, _CH)

    degp = _sc_degree(ei3, n, npad)
    d0w = _widen16(degp[0, 0], npad)
    d1w = _widen16(degp[1, 0], npad)
    xw = jnp.pad(x, ((0, npad - n), (0, _H - x.shape[1]))).reshape(r, 128)
    dis16w, u1w = _tc_prep(d0w, d1w, xw)

    agg1 = _sc_scatter(ei3, u1w.reshape(npad, _H), n)
    bigW1, b1w = _blockdiag(W1, b1)
    u2w = _tc_layer(agg1[0].reshape(r, 128), agg1[1].reshape(r, 128),
                    u1w, dis16w, bigW1, b1w)

    agg2 = _sc_scatter(ei3, u2w.reshape(npad, _H), n)
    bigW2, b2w = _blockdiag(W2, b2)
    batp = jnp.pad(batch, (0, npad - n), constant_values=_G + 1)
    bat8 = batp.reshape(r, 8).T
    efold = (jnp.arange(128)[:, None] % _H ==
             jnp.arange(_H)[None, :]).astype(jnp.float32)
    out = _tc_final(agg2[0].reshape(r, 128), agg2[1].reshape(r, 128),
                    u2w, dis16w, bigW2, b2w, bat8, efold,
                    fc1_W, fc1_b.reshape(1, -1), fc2_W, fc2_b.reshape(1, 1))
    return out


# speedup vs baseline: 63.3411x; 1.0046x over previous
import functools

import jax
import jax.numpy as jnp
from jax import lax
from jax.experimental import pallas as pl
from jax.experimental.pallas import tpu as pltpu
from jax.experimental.pallas import tpu_sc as plsc

_CH = 256
_BLK = 4
_NW = 32
_G = 64
_H = 16
_GRID = 49


def _sc_degree(ei3, n, npad):
    _, rows, ch = ei3.shape
    nbt = rows // _BLK
    nbw = (nbt + _NW - 1) // _NW
    nbw = nbw + (nbw % 2)
    mesh = plsc.VectorSubcoreMesh(core_axis_name="c", subcore_axis_name="s")

    @functools.partial(
        pl.kernel,
        out_type=jax.ShapeDtypeStruct((2, 1, npad), jnp.float32),
        mesh=mesh,
        scratch_types=[
            pltpu.VMEM((_BLK, ch), jnp.int32),
            pltpu.VMEM((_BLK, ch), jnp.int32),
            pltpu.VMEM((ch,), jnp.float32),
            pltpu.VMEM_SHARED((n,), jnp.float32),
            pltpu.SemaphoreType.DMA,
            pltpu.SemaphoreType.DMA,
        ],
        compiler_params=pltpu.CompilerParams(use_tc_tiling_on_sc=False),
    )
    def deg_kernel(ei_hbm, ones_hbm, z_hbm, out_hbm,
                   colv0, colv1, onesv, acc, sem0, sem1):
        cid = lax.axis_index("c")
        sid = lax.axis_index("s")
        wid = cid * 16 + sid
        col_hbm = ei_hbm.at[1]

        @pl.when(sid == 0)
        def _zero():
            pltpu.sync_copy(z_hbm, acc)

        pltpu.sync_copy(ones_hbm, onesv)
        plsc.subcore_barrier()

        def fire(colv, sem):
            for j in range(_BLK):
                pltpu.async_copy(onesv, acc.at[colv.at[j]], sem, add=True)

        def wait(colv, sem):
            for j in range(_BLK):
                pltpu.make_async_copy(onesv, acc.at[colv.at[j]], sem).wait()

        def body(g, carry):
            be = (2 * g) * _NW + wid
            bo = be + _NW

            @pl.when((bo - 2 * _NW >= 0) & (bo - 2 * _NW < nbt))
            def _():
                wait(colv1, sem1)

            @pl.when(be < nbt)
            def _():
                pltpu.sync_copy(col_hbm.at[pl.ds(be * _BLK, _BLK)], colv0)
                fire(colv0, sem0)

            @pl.when(bo < nbt)
            def _():
                pltpu.sync_copy(col_hbm.at[pl.ds(bo * _BLK, _BLK)], colv1)
                fire(colv1, sem1)

            @pl.when(be < nbt)
            def _():
                wait(colv0, sem0)

            return carry

        lax.fori_loop(0, nbw // 2, body, 0)
        blast = (nbw - 1) * _NW + wid

        @pl.when(blast < nbt)
        def _():
            wait(colv1, sem1)

        plsc.subcore_barrier()

        @pl.when(sid == 0)
        def _out():
            pltpu.sync_copy(acc, out_hbm.at[cid, 0, pl.ds(0, n)])

    return deg_kernel(ei3, jnp.ones((ch,), jnp.float32),
                      jnp.zeros((n,), jnp.float32))


def _sc_scatter(ei3, u, n):
    _, rows, ch = ei3.shape
    npad, f = u.shape
    nbt = rows // _BLK
    nbw = (nbt + _NW - 1) // _NW
    nbw = nbw + (nbw % 2)
    hb = _BLK // 2
    mesh = plsc.VectorSubcoreMesh(core_axis_name="c", subcore_axis_name="s")

    @functools.partial(
        pl.kernel,
        out_type=jax.ShapeDtypeStruct((2, npad, f), jnp.float32),
        mesh=mesh,
        scratch_types=[
            pltpu.VMEM((_BLK, ch), jnp.int32),
            pltpu.VMEM((_BLK, ch), jnp.int32),
            pltpu.VMEM((_BLK, ch), jnp.int32),
            pltpu.VMEM((_BLK, ch), jnp.int32),
            pltpu.VMEM((_BLK // 2 * ch, f), jnp.float32),
            pltpu.VMEM((_BLK // 2 * ch, f), jnp.float32),
            pltpu.VMEM_SHARED((n, f), jnp.float32),
            pltpu.SemaphoreType.DMA,
            pltpu.SemaphoreType.DMA,
            pltpu.SemaphoreType.DMA,
            pltpu.SemaphoreType.DMA,
        ],
        compiler_params=pltpu.CompilerParams(use_tc_tiling_on_sc=False),
    )
    def scat_kernel(ei_hbm, u_hbm, z_hbm, out_hbm,
                    rowv0, colv0, rowv1, colv1, data0, data1, acc,
                    semg0, semg1, sems0, sems1):
        hb_ = _BLK // 2
        cid = lax.axis_index("c")
        sid = lax.axis_index("s")
        wid = cid * 16 + sid
        row_hbm = ei_hbm.at[0]
        col_hbm = ei_hbm.at[1]

        @pl.when(sid == 0)
        def _zero():
            pltpu.sync_copy(z_hbm, acc)

        plsc.subcore_barrier()

        def load(b, rowv, colv):
            pltpu.sync_copy(row_hbm.at[pl.ds(b * _BLK, _BLK)], rowv)
            pltpu.sync_copy(col_hbm.at[pl.ds(b * _BLK, _BLK)], colv)

        def fire_g(rowv, j0, datav, sem):
            for j in range(hb_):
                pltpu.async_copy(u_hbm.at[rowv.at[j0 + j]],
                                 datav.at[pl.ds(j * ch, ch)], sem)

        def wait_g(rowv, j0, datav, sem):
            for j in range(hb_):
                pltpu.make_async_copy(
                    u_hbm.at[rowv.at[j0 + j]],
                    datav.at[pl.ds(j * ch, ch)], sem).wait()

        def fire_s(colv, j0, datav, sem):
            for j in range(hb_):
                pltpu.async_copy(datav.at[pl.ds(j * ch, ch)],
                                 acc.at[colv.at[j0 + j]], sem, add=True)

        def wait_s(colv, j0, datav, sem):
            for j in range(hb_):
                pltpu.make_async_copy(
                    datav.at[pl.ds(j * ch, ch)],
                    acc.at[colv.at[j0 + j]], sem).wait()

        def body(g, carry):
            be = (2 * g) * _NW + wid
            bo = be + _NW
            bp = bo - 2 * _NW
            cp = (bp >= 0) & (bp < nbt)
            ce = be < nbt
            co = bo < nbt

            @pl.when(cp)
            def _():
                wait_s(colv1, 0, data0, sems0)

            @pl.when(ce)
            def _():
                load(be, rowv0, colv0)
                fire_g(rowv0, 0, data0, semg0)

            @pl.when(cp)
            def _():
                wait_s(colv1, hb_, data1, sems1)

            @pl.when(ce)
            def _():
                fire_g(rowv0, hb_, data1, semg1)
                wait_g(rowv0, 0, data0, semg0)
                fire_s(colv0, 0, data0, sems0)

            @pl.when(co)
            def _():
                load(bo, rowv1, colv1)

            @pl.when(ce)
            def _():
                wait_g(rowv0, hb_, data1, semg1)
                fire_s(colv0, hb_, data1, sems1)

            @pl.when(co)
            def _():
                wait_s(colv0, 0, data0, sems0)
                fire_g(rowv1, 0, data0, semg0)
                wait_g(rowv1, 0, data0, semg0)
                fire_s(colv1, 0, data0, sems0)
                wait_s(colv0, hb_, data1, sems1)
                fire_g(rowv1, hb_, data1, semg1)
                wait_g(rowv1, hb_, data1, semg1)
                fire_s(colv1, hb_, data1, sems1)

            return carry

        lax.fori_loop(0, nbw // 2, body, 0)
        bo_last = (nbw - 1) * _NW + wid
        ce_last = (bo_last - _NW) < nbt
        co_last = bo_last < nbt

        @pl.when(co_last)
        def _():
            wait_s(colv1, 0, data0, sems0)
            wait_s(colv1, hb_, data1, sems1)

        @pl.when(ce_last & jnp.logical_not(co_last))
        def _():
            wait_s(colv0, 0, data0, sems0)
            wait_s(colv0, hb_, data1, sems1)

        plsc.subcore_barrier()

        @pl.when(sid == 0)
        def _out():
            pltpu.sync_copy(acc, out_hbm.at[cid, pl.ds(0, n)])

    return scat_kernel(ei3, u, jnp.zeros((n, f), jnp.float32))


def _tc_prep(d0w, d1w, xw):
    r = d0w.shape[0]
    rb = r // _GRID

    def body(d0_ref, d1_ref, x_ref, dis_ref, u1_ref):
        dis = lax.rsqrt(d0_ref[...] + d1_ref[...] + 1.0)
        dis_ref[...] = dis
        u1_ref[...] = x_ref[...] * dis

    return pl.pallas_call(
        body,
        grid=(_GRID,),
        in_specs=[pl.BlockSpec((rb, 128), lambda i: (i, 0))] * 3,
        out_specs=[pl.BlockSpec((rb, 128), lambda i: (i, 0))] * 2,
        out_shape=[jax.ShapeDtypeStruct((r, 128), jnp.float32)] * 2,
    )(d0w, d1w, xw)


def _tc_layer(a0w, a1w, uw, dis16w, bigW, biasw):
    r = a0w.shape[0]
    rb = r // _GRID

    def body(a0_ref, a1_ref, u_ref, dis_ref, w_ref, b_ref, o_ref):
        dis = dis_ref[...]
        t = dis * (a0_ref[...] + a1_ref[...] + u_ref[...])
        h = jnp.maximum(
            lax.dot_general(t, w_ref[...], (((1,), (0,)), ((), ())),
                            preferred_element_type=jnp.float32) + b_ref[...],
            0.0)
        o_ref[...] = h * dis

    return pl.pallas_call(
        body,
        grid=(_GRID,),
        in_specs=[
            pl.BlockSpec((rb, 128), lambda i: (i, 0)),
            pl.BlockSpec((rb, 128), lambda i: (i, 0)),
            pl.BlockSpec((rb, 128), lambda i: (i, 0)),
            pl.BlockSpec((rb, 128), lambda i: (i, 0)),
            pl.BlockSpec((128, 128), lambda i: (0, 0)),
            pl.BlockSpec((1, 128), lambda i: (0, 0)),
        ],
        out_specs=pl.BlockSpec((rb, 128), lambda i: (i, 0)),
        out_shape=jax.ShapeDtypeStruct((r, 128), jnp.float32),
    )(a0w, a1w, uw, dis16w, bigW, biasw)


def _tc_final(a0w, a1w, u2w, dis16w, bigW, biasw, bat8, efold,
              f1W, f1b, f2W, f2b):
    r = a0w.shape[0]
    rb = r // _GRID

    def body(a0_ref, a1_ref, u_ref, dis_ref, w_ref, b_ref, bat_ref,
             ef_ref, f1w_ref, f1b_ref, f2w_ref, f2b_ref, out_ref,
             pooled, cnts):
        i = pl.program_id(0)

        @pl.when(i == 0)
        def _init():
            pooled[...] = jnp.zeros_like(pooled)
            cnts[...] = jnp.zeros_like(cnts)

        dis = dis_ref[...]
        t = dis * (a0_ref[...] + a1_ref[...] + u_ref[...])
        h2 = jnp.maximum(
            lax.dot_general(t, w_ref[...], (((1,), (0,)), ((), ())),
                            preferred_element_type=jnp.float32) + b_ref[...],
            0.0)
        h2 = jnp.where(jnp.abs(h2) < 1e30, h2, 0.0)

        gcol = lax.broadcasted_iota(jnp.int32, (_G, rb), 0)
        slot = lax.broadcasted_iota(jnp.int32, (_G, 128), 1) // _H
        ones = jnp.ones((rb, 128), jnp.float32)
        psum = jnp.zeros((_G, 128), jnp.float32)
        csum = jnp.zeros((_G, 128), jnp.float32)
        for a in range(8):
            ba = bat_ref[a:a + 1, :]
            oh = (jnp.broadcast_to(ba, (_G, rb)) == gcol).astype(jnp.float32)
            m = (slot == a).astype(jnp.float32)
            psum += lax.dot_general(oh, h2, (((1,), (0,)), ((), ())),
                                    preferred_element_type=jnp.float32) * m
            csum += lax.dot_general(oh, ones, (((1,), (0,)), ((), ())),
                                    preferred_element_type=jnp.float32) * m

        ef = ef_ref[...]
        pooled[...] += lax.dot_general(psum, ef, (((1,), (0,)), ((), ())),
                                       preferred_element_type=jnp.float32)
        cnts[...] += lax.dot_general(csum, ef, (((1,), (0,)), ((), ())),
                                     preferred_element_type=jnp.float32)

        @pl.when(i == pl.num_programs(0) - 1)
        def _fin():
            mean = pooled[...] / jnp.maximum(cnts[...], 1.0)
            hm = jnp.maximum(
                lax.dot_general(mean, f1w_ref[...], (((1,), (0,)), ((), ())),
                                preferred_element_type=jnp.float32)
                + f1b_ref[...], 0.0)
            out_ref[...] = lax.dot_general(
                hm, f2w_ref[...], (((1,), (0,)), ((), ())),
                preferred_element_type=jnp.float32) + f2b_ref[...]

    return pl.pallas_call(
        body,
        grid=(_GRID,),
        in_specs=[
            pl.BlockSpec((rb, 128), lambda i: (i, 0)),
            pl.BlockSpec((rb, 128), lambda i: (i, 0)),
            pl.BlockSpec((rb, 128), lambda i: (i, 0)),
            pl.BlockSpec((rb, 128), lambda i: (i, 0)),
            pl.BlockSpec((128, 128), lambda i: (0, 0)),
            pl.BlockSpec((1, 128), lambda i: (0, 0)),
            pl.BlockSpec((8, rb), lambda i: (0, i)),
            pl.BlockSpec((128, _H), lambda i: (0, 0)),
            pl.BlockSpec((_H, _H), lambda i: (0, 0)),
            pl.BlockSpec((1, _H), lambda i: (0, 0)),
            pl.BlockSpec((_H, 1), lambda i: (0, 0)),
            pl.BlockSpec((1, 1), lambda i: (0, 0)),
        ],
        out_specs=pl.BlockSpec((_G, 1), lambda i: (0, 0)),
        out_shape=jax.ShapeDtypeStruct((_G, 1), jnp.float32),
        scratch_shapes=[
            pltpu.VMEM((_G, _H), jnp.float32),
            pltpu.VMEM((_G, _H), jnp.float32),
        ],
    )(a0w, a1w, u2w, dis16w, bigW, biasw, bat8, efold, f1W, f1b, f2W, f2b)


def _widen16(v, npad):
    return jnp.broadcast_to(v[:, None], (npad, _H)).reshape(npad // 8, 128)


def _blockdiag(W, b):
    Wp = jnp.zeros((_H, _H), jnp.float32).at[:W.shape[0], :].set(W)
    bigW = jnp.kron(jnp.eye(8, dtype=jnp.float32), Wp)
    biasw = jnp.tile(b, 8).reshape(1, 128)
    return bigW, biasw


def kernel(x, edge_index, batch, W1, b1, W2, b2, fc1_W, fc1_b, fc2_W, fc2_b):
    n = x.shape[0]
    e = edge_index.shape[1]
    npad = ((n + 1023) // 1024) * 1024
    r = npad // 8

    ei3 = edge_index.reshape(2, e // _CH, _CH)

    degp = _sc_degree(ei3, n, npad)
    d0w = _widen16(degp[0, 0], npad)
    d1w = _widen16(degp[1, 0], npad)
    xw = jnp.pad(x, ((0, npad - n), (0, _H - x.shape[1]))).reshape(r, 128)
    dis16w, u1w = _tc_prep(d0w, d1w, xw)

    agg1 = _sc_scatter(ei3, u1w.reshape(npad, _H), n)
    bigW1, b1w = _blockdiag(W1, b1)
    u2w = _tc_layer(agg1[0].reshape(r, 128), agg1[1].reshape(r, 128),
                    u1w, dis16w, bigW1, b1w)

    agg2 = _sc_scatter(ei3, u2w.reshape(npad, _H), n)
    bigW2, b2w = _blockdiag(W2, b2)
    batp = jnp.pad(batch, (0, npad - n), constant_values=_G + 1)
    bat8 = batp.reshape(r, 8).T
    efold = (jnp.arange(128)[:, None] % _H ==
             jnp.arange(_H)[None, :]).astype(jnp.float32)
    out = _tc_final(agg2[0].reshape(r, 128), agg2[1].reshape(r, 128),
                    u2w, dis16w, bigW2, b2w, bat8, efold,
                    fc1_W, fc1_b.reshape(1, -1), fc2_W, fc2_b.reshape(1, 1))
    return out


# speedup vs baseline: 64.4916x; 1.0182x over previous
import functools

import jax
import jax.numpy as jnp
from jax import lax
from jax.experimental import pallas as pl
from jax.experimental.pallas import tpu as pltpu
from jax.experimental.pallas import tpu_sc as plsc

_CH = 256
_BLK = 4
_NW = 32
_G = 64
_H = 16
_GRID = 49


def _sc_degree(ei3, n, npad):
    _, rows, ch = ei3.shape
    nbt = rows // _BLK
    nbw = (nbt + _NW - 1) // _NW
    nbw = nbw + (nbw % 2)
    mesh = plsc.VectorSubcoreMesh(core_axis_name="c", subcore_axis_name="s")

    @functools.partial(
        pl.kernel,
        out_type=jax.ShapeDtypeStruct((2, 1, npad), jnp.float32),
        mesh=mesh,
        scratch_types=[
            pltpu.VMEM((_BLK, ch), jnp.int32),
            pltpu.VMEM((_BLK, ch), jnp.int32),
            pltpu.VMEM((ch,), jnp.float32),
            pltpu.VMEM_SHARED((n,), jnp.float32),
            pltpu.SemaphoreType.DMA,
            pltpu.SemaphoreType.DMA,
        ],
        compiler_params=pltpu.CompilerParams(use_tc_tiling_on_sc=False),
    )
    def deg_kernel(ei_hbm, ones_hbm, z_hbm, out_hbm,
                   colv0, colv1, onesv, acc, sem0, sem1):
        cid = lax.axis_index("c")
        sid = lax.axis_index("s")
        wid = cid * 16 + sid
        col_hbm = ei_hbm.at[1]

        @pl.when(sid == 0)
        def _zero():
            pltpu.sync_copy(z_hbm, acc)

        pltpu.sync_copy(ones_hbm, onesv)
        plsc.subcore_barrier()

        def fire(colv, sem):
            for j in range(_BLK):
                pltpu.async_copy(onesv, acc.at[colv.at[j]], sem, add=True)

        def wait(colv, sem):
            for j in range(_BLK):
                pltpu.make_async_copy(onesv, acc.at[colv.at[j]], sem).wait()

        def body(g, carry):
            be = (2 * g) * _NW + wid
            bo = be + _NW

            @pl.when((bo - 2 * _NW >= 0) & (bo - 2 * _NW < nbt))
            def _():
                wait(colv1, sem1)

            @pl.when(be < nbt)
            def _():
                pltpu.sync_copy(col_hbm.at[pl.ds(be * _BLK, _BLK)], colv0)
                fire(colv0, sem0)

            @pl.when(bo < nbt)
            def _():
                pltpu.sync_copy(col_hbm.at[pl.ds(bo * _BLK, _BLK)], colv1)
                fire(colv1, sem1)

            @pl.when(be < nbt)
            def _():
                wait(colv0, sem0)

            return carry

        lax.fori_loop(0, nbw // 2, body, 0)
        blast = (nbw - 1) * _NW + wid

        @pl.when(blast < nbt)
        def _():
            wait(colv1, sem1)

        plsc.subcore_barrier()

        @pl.when(sid == 0)
        def _out():
            pltpu.sync_copy(acc, out_hbm.at[cid, 0, pl.ds(0, n)])

    return deg_kernel(ei3, jnp.ones((ch,), jnp.float32),
                      jnp.zeros((n,), jnp.float32))


def _sc_scatter(ei3, u, n):
    _, rows, ch = ei3.shape
    npad, f = u.shape
    nbt = rows // _BLK
    nbw = (nbt + _NW - 1) // _NW
    nbw = nbw + (nbw % 2)
    hb = _BLK // 2
    mesh = plsc.VectorSubcoreMesh(core_axis_name="c", subcore_axis_name="s")

    @functools.partial(
        pl.kernel,
        out_type=jax.ShapeDtypeStruct((2, npad, f), jnp.float32),
        mesh=mesh,
        scratch_types=[
            pltpu.VMEM((_BLK, ch), jnp.int32),
            pltpu.VMEM((_BLK, ch), jnp.int32),
            pltpu.VMEM((_BLK, ch), jnp.int32),
            pltpu.VMEM((_BLK, ch), jnp.int32),
            pltpu.VMEM((_BLK // 2 * ch, f), jnp.float32),
            pltpu.VMEM((_BLK // 2 * ch, f), jnp.float32),
            pltpu.VMEM_SHARED((n, f), jnp.float32),
            pltpu.SemaphoreType.DMA,
            pltpu.SemaphoreType.DMA,
            pltpu.SemaphoreType.DMA,
            pltpu.SemaphoreType.DMA,
        ],
        compiler_params=pltpu.CompilerParams(use_tc_tiling_on_sc=False),
    )
    def scat_kernel(ei_hbm, u_hbm, z_hbm, out_hbm,
                    rowv0, colv0, rowv1, colv1, data0, data1, acc,
                    semg0, semg1, sems0, sems1):
        hb_ = _BLK // 2
        cid = lax.axis_index("c")
        sid = lax.axis_index("s")
        wid = cid * 16 + sid
        row_hbm = ei_hbm.at[0]
        col_hbm = ei_hbm.at[1]

        @pl.when(sid == 0)
        def _zero():
            pltpu.sync_copy(z_hbm, acc)

        plsc.subcore_barrier()

        def load(b, rowv, colv):
            pltpu.sync_copy(row_hbm.at[pl.ds(b * _BLK, _BLK)], rowv)
            pltpu.sync_copy(col_hbm.at[pl.ds(b * _BLK, _BLK)], colv)

        def fire_g(rowv, j0, datav, sem):
            for j in range(hb_):
                pltpu.async_copy(u_hbm.at[rowv.at[j0 + j]],
                                 datav.at[pl.ds(j * ch, ch)], sem)

        def wait_g(rowv, j0, datav, sem):
            for j in range(hb_):
                pltpu.make_async_copy(
                    u_hbm.at[rowv.at[j0 + j]],
                    datav.at[pl.ds(j * ch, ch)], sem).wait()

        def fire_s(colv, j0, datav, sem):
            for j in range(hb_):
                pltpu.async_copy(datav.at[pl.ds(j * ch, ch)],
                                 acc.at[colv.at[j0 + j]], sem, add=True)

        def wait_s(colv, j0, datav, sem):
            for j in range(hb_):
                pltpu.make_async_copy(
                    datav.at[pl.ds(j * ch, ch)],
                    acc.at[colv.at[j0 + j]], sem).wait()

        def body(g, carry):
            be = (2 * g) * _NW + wid
            bo = be + _NW
            bp = bo - 2 * _NW
            cp = (bp >= 0) & (bp < nbt)
            ce = be < nbt
            co = bo < nbt

            @pl.when(cp)
            def _():
                wait_s(colv1, 0, data0, sems0)

            @pl.when(ce)
            def _():
                load(be, rowv0, colv0)
                fire_g(rowv0, 0, data0, semg0)

            @pl.when(cp)
            def _():
                wait_s(colv1, hb_, data1, sems1)

            @pl.when(ce)
            def _():
                fire_g(rowv0, hb_, data1, semg1)
                wait_g(rowv0, 0, data0, semg0)
                fire_s(colv0, 0, data0, sems0)

            @pl.when(co)
            def _():
                load(bo, rowv1, colv1)

            @pl.when(ce)
            def _():
                wait_g(rowv0, hb_, data1, semg1)
                fire_s(colv0, hb_, data1, sems1)

            @pl.when(co)
            def _():
                wait_s(colv0, 0, data0, sems0)
                fire_g(rowv1, 0, data0, semg0)
                wait_g(rowv1, 0, data0, semg0)
                fire_s(colv1, 0, data0, sems0)
                wait_s(colv0, hb_, data1, sems1)
                fire_g(rowv1, hb_, data1, semg1)
                wait_g(rowv1, hb_, data1, semg1)
                fire_s(colv1, hb_, data1, sems1)

            return carry

        lax.fori_loop(0, nbw // 2, body, 0)
        bo_last = (nbw - 1) * _NW + wid
        ce_last = (bo_last - _NW) < nbt
        co_last = bo_last < nbt

        @pl.when(co_last)
        def _():
            wait_s(colv1, 0, data0, sems0)
            wait_s(colv1, hb_, data1, sems1)

        @pl.when(ce_last & jnp.logical_not(co_last))
        def _():
            wait_s(colv0, 0, data0, sems0)
            wait_s(colv0, hb_, data1, sems1)

        plsc.subcore_barrier()

        @pl.when(sid == 0)
        def _out():
            pltpu.sync_copy(acc, out_hbm.at[cid, pl.ds(0, n)])

    return scat_kernel(ei3, u, jnp.zeros((n, f), jnp.float32))


def _tc_prep(dw, xw):
    r = dw.shape[0]
    rb = r // _GRID

    def body(d_ref, x_ref, dis_ref, u1_ref):
        dis = lax.rsqrt(d_ref[...] + 1.0)
        dis_ref[...] = dis
        u1_ref[...] = x_ref[...] * dis

    return pl.pallas_call(
        body,
        grid=(_GRID,),
        in_specs=[pl.BlockSpec((rb, 128), lambda i: (i, 0))] * 2,
        out_specs=[pl.BlockSpec((rb, 128), lambda i: (i, 0))] * 2,
        out_shape=[jax.ShapeDtypeStruct((r, 128), jnp.float32)] * 2,
    )(dw, xw)


def _tc_layer(a0w, a1w, uw, dis16w, bigW, biasw):
    r = a0w.shape[0]
    rb = r // _GRID

    def body(a0_ref, a1_ref, u_ref, dis_ref, w_ref, b_ref, o_ref):
        dis = dis_ref[...]
        t = dis * (a0_ref[...] + a1_ref[...] + u_ref[...])
        h = jnp.maximum(
            lax.dot_general(t, w_ref[...], (((1,), (0,)), ((), ())),
                            preferred_element_type=jnp.float32) + b_ref[...],
            0.0)
        o_ref[...] = h * dis

    return pl.pallas_call(
        body,
        grid=(_GRID,),
        in_specs=[
            pl.BlockSpec((rb, 128), lambda i: (i, 0)),
            pl.BlockSpec((rb, 128), lambda i: (i, 0)),
            pl.BlockSpec((rb, 128), lambda i: (i, 0)),
            pl.BlockSpec((rb, 128), lambda i: (i, 0)),
            pl.BlockSpec((128, 128), lambda i: (0, 0)),
            pl.BlockSpec((1, 128), lambda i: (0, 0)),
        ],
        out_specs=pl.BlockSpec((rb, 128), lambda i: (i, 0)),
        out_shape=jax.ShapeDtypeStruct((r, 128), jnp.float32),
    )(a0w, a1w, uw, dis16w, bigW, biasw)


def _tc_final(a0w, a1w, u2w, dis16w, bigW, biasw, bat8, efold,
              f1W, f1b, f2W, f2b):
    r = a0w.shape[0]
    rb = r // _GRID

    def body(a0_ref, a1_ref, u_ref, dis_ref, w_ref, b_ref, bat_ref,
             ef_ref, f1w_ref, f1b_ref, f2w_ref, f2b_ref, out_ref,
             pooled, cnts):
        i = pl.program_id(0)

        @pl.when(i == 0)
        def _init():
            pooled[...] = jnp.zeros_like(pooled)
            cnts[...] = jnp.zeros_like(cnts)

        dis = dis_ref[...]
        t = dis * (a0_ref[...] + a1_ref[...] + u_ref[...])
        h2 = jnp.maximum(
            lax.dot_general(t, w_ref[...], (((1,), (0,)), ((), ())),
                            preferred_element_type=jnp.float32) + b_ref[...],
            0.0)
        h2 = jnp.where(jnp.abs(h2) < 1e30, h2, 0.0)

        gcol = lax.broadcasted_iota(jnp.int32, (_G, rb), 0)
        slot = lax.broadcasted_iota(jnp.int32, (_G, 128), 1) // _H
        ones = jnp.ones((rb, 128), jnp.float32)
        psum = jnp.zeros((_G, 128), jnp.float32)
        csum = jnp.zeros((_G, 128), jnp.float32)
        for a in range(8):
            ba = bat_ref[a:a + 1, :]
            oh = (jnp.broadcast_to(ba, (_G, rb)) == gcol).astype(jnp.float32)
            m = (slot == a).astype(jnp.float32)
            psum += lax.dot_general(oh, h2, (((1,), (0,)), ((), ())),
                                    preferred_element_type=jnp.float32) * m
            csum += lax.dot_general(oh, ones, (((1,), (0,)), ((), ())),
                                    preferred_element_type=jnp.float32) * m

        ef = ef_ref[...]
        pooled[...] += lax.dot_general(psum, ef, (((1,), (0,)), ((), ())),
                                       preferred_element_type=jnp.float32)
        cnts[...] += lax.dot_general(csum, ef, (((1,), (0,)), ((), ())),
                                     preferred_element_type=jnp.float32)

        @pl.when(i == pl.num_programs(0) - 1)
        def _fin():
            mean = pooled[...] / jnp.maximum(cnts[...], 1.0)
            hm = jnp.maximum(
                lax.dot_general(mean, f1w_ref[...], (((1,), (0,)), ((), ())),
                                preferred_element_type=jnp.float32)
                + f1b_ref[...], 0.0)
            out_ref[...] = lax.dot_general(
                hm, f2w_ref[...], (((1,), (0,)), ((), ())),
                preferred_element_type=jnp.float32) + f2b_ref[...]

    return pl.pallas_call(
        body,
        grid=(_GRID,),
        in_specs=[
            pl.BlockSpec((rb, 128), lambda i: (i, 0)),
            pl.BlockSpec((rb, 128), lambda i: (i, 0)),
            pl.BlockSpec((rb, 128), lambda i: (i, 0)),
            pl.BlockSpec((rb, 128), lambda i: (i, 0)),
            pl.BlockSpec((128, 128), lambda i: (0, 0)),
            pl.BlockSpec((1, 128), lambda i: (0, 0)),
            pl.BlockSpec((8, rb), lambda i: (0, i)),
            pl.BlockSpec((128, _H), lambda i: (0, 0)),
            pl.BlockSpec((_H, _H), lambda i: (0, 0)),
            pl.BlockSpec((1, _H), lambda i: (0, 0)),
            pl.BlockSpec((_H, 1), lambda i: (0, 0)),
            pl.BlockSpec((1, 1), lambda i: (0, 0)),
        ],
        out_specs=pl.BlockSpec((_G, 1), lambda i: (0, 0)),
        out_shape=jax.ShapeDtypeStruct((_G, 1), jnp.float32),
        scratch_shapes=[
            pltpu.VMEM((_G, _H), jnp.float32),
            pltpu.VMEM((_G, _H), jnp.float32),
        ],
    )(a0w, a1w, u2w, dis16w, bigW, biasw, bat8, efold, f1W, f1b, f2W, f2b)


def _widen16(v, npad):
    return jnp.broadcast_to(v[:, None], (npad, _H)).reshape(npad // 8, 128)


def _blockdiag(W, b):
    Wp = jnp.zeros((_H, _H), jnp.float32).at[:W.shape[0], :].set(W)
    bigW = jnp.kron(jnp.eye(8, dtype=jnp.float32), Wp)
    biasw = jnp.tile(b, 8).reshape(1, 128)
    return bigW, biasw


def kernel(x, edge_index, batch, W1, b1, W2, b2, fc1_W, fc1_b, fc2_W, fc2_b):
    n = x.shape[0]
    e = edge_index.shape[1]
    npad = ((n + 1023) // 1024) * 1024
    r = npad // 8

    ei3 = edge_index.reshape(2, e // _CH, _CH)

    degp = _sc_degree(ei3, n, npad)
    dw = _widen16(degp[0, 0] + degp[1, 0], npad)
    xw = jnp.pad(x, ((0, npad - n), (0, _H - x.shape[1]))).reshape(r, 128)
    dis16w, u1w = _tc_prep(dw, xw)

    agg1 = _sc_scatter(ei3, u1w.reshape(npad, _H), n)
    bigW1, b1w = _blockdiag(W1, b1)
    u2w = _tc_layer(agg1[0].reshape(r, 128), agg1[1].reshape(r, 128),
                    u1w, dis16w, bigW1, b1w)

    agg2 = _sc_scatter(ei3, u2w.reshape(npad, _H), n)
    bigW2, b2w = _blockdiag(W2, b2)
    batp = jnp.pad(batch, (0, npad - n), constant_values=_G + 1)
    bat8 = batp.reshape(r, 8).T
    efold = (jnp.arange(128)[:, None] % _H ==
             jnp.arange(_H)[None, :]).astype(jnp.float32)
    out = _tc_final(agg2[0].reshape(r, 128), agg2[1].reshape(r, 128),
                    u2w, dis16w, bigW2, b2w, bat8, efold,
                    fc1_W, fc1_b.reshape(1, -1), fc2_W, fc2_b.reshape(1, 1))
    return out
